# baseline (speedup 1.0000x reference)
import functools

import jax
import jax.numpy as jnp
from jax import lax
from jax.experimental import pallas as pl
from jax.experimental.pallas import tpu as pltpu
from jax.experimental.pallas import tpu_sc as plsc

N_NODES = 100000
N_EDGES = 6400000
N_STEPS = 5

NC = 2
NS = 16
N_TILES = NC * NS

SLICE = 6400
NP = NS * SLICE

CHUNK_R = 8
CHUNK_C = 128
CHUNK = CHUNK_R * CHUNK_C
N_CHUNKS = N_EDGES // CHUNK
N_UNIF = N_CHUNKS // N_TILES
N_EXTRA = N_CHUNKS - N_UNIF * N_TILES
NBUF = 4

_PIECES = [(0, 2048), (2048, 2048), (4096, 2048), (6144, 256)]

_mesh = plsc.VectorSubcoreMesh(core_axis_name="c", subcore_axis_name="s")
_params = pltpu.CompilerParams(needs_layout_passes=False)


def _step_body(x_in, ei, dstf, prb, pa_out, pb_out,
               x_buf, stage,
               src_v0, src_v1, src_v2, src_v3, dst_v0, dst_v1, dst_v2, dst_v3,
               prb_v0, prb_v1, prb_v2, prb_v3, val_v0, val_v1, val_v2, val_v3,
               sem_in0, sem_in1, sem_in2, sem_in3, sem_sc, acc):
    cid = lax.axis_index("c")
    sid = lax.axis_index("s")
    tid = cid * NS + sid
    base = sid * SLICE
    sems = [sem_in0, sem_in1, sem_in2, sem_in3]
    src_vs = [src_v0, src_v1, src_v2, src_v3]
    dst_vs = [dst_v0, dst_v1, dst_v2, dst_v3]
    prb_vs = [prb_v0, prb_v1, prb_v2, prb_v3]
    val_vs = [val_v0, val_v1, val_v2, val_v3]

    pltpu.sync_copy(x_in, x_buf)

    zv = jnp.zeros((16,), jnp.float32)

    def zero_body(j, _):
        stage[pl.ds(j * 16, 16)] = zv
        return _

    lax.fori_loop(0, 128, zero_body, 0)
    for off, sz in _PIECES:
        pltpu.sync_copy(stage.at[pl.ds(0, sz)], acc.at[pl.ds(base + off, sz)])
    plsc.subcore_barrier()

    def issue(j, b):
        g = tid + N_TILES * j
        pltpu.async_copy(ei.at[0, g], src_vs[b], sems[b])
        pltpu.async_copy(dstf.at[pl.ds(g * CHUNK, CHUNK)], dst_vs[b], sems[b])
        pltpu.async_copy(prb.at[g], prb_vs[b], sems[b])

    def wait_in(j, b):
        g = tid + N_TILES * j
        pltpu.make_async_copy(ei.at[0, g], src_vs[b], sems[b]).wait()
        pltpu.make_async_copy(dstf.at[pl.ds(g * CHUNK, CHUNK)], dst_vs[b],
                              sems[b]).wait()
        pltpu.make_async_copy(prb.at[g], prb_vs[b], sems[b]).wait()

    def gather(b):
        for r in range(CHUNK_R):
            for j in range(CHUNK_C // 16):
                idx = src_vs[b][r, pl.ds(j * 16, 16)]
                v = (plsc.load_gather(x_buf, [idx])
                     * prb_vs[b][r, pl.ds(j * 16, 16)])
                val_vs[b][pl.ds(r * CHUNK_C + j * 16, 16)] = v

    def fire(b):
        pltpu.async_copy(val_vs[b], acc.at[dst_vs[b]], sem_sc, add=True)

    def drain(b):
        pltpu.make_async_copy(dstf.at[pl.ds(0, CHUNK)], dst_vs[b],
                              sem_sc).wait()

    issue(0, 0)
    issue(1, 1)

    def loop_body(t, carry):
        for u in range(NBUF):
            j = t * NBUF + u
            wait_in(j, u)
            gather(u)
            if u <= 1:
                @pl.when(t >= 1)
                def _():
                    drain((u + 2) % NBUF)
            else:
                drain(u - 2)
            issue(j + 2, (u + 2) % NBUF)
            fire(u)
        return carry

    n_main = (N_UNIF - 3) // NBUF
    lax.fori_loop(0, n_main, loop_body, 0)
    for j in range(n_main * NBUF, N_UNIF):
        u = j % NBUF
        wait_in(j, u)
        gather(u)
        drain((u + 2) % NBUF)
        if j + 2 < N_UNIF:
            issue(j + 2, (j + 2) % NBUF)
        fire(u)
    drain((N_UNIF - 2) % NBUF)
    drain((N_UNIF - 1) % NBUF)

    @pl.when(tid < N_EXTRA)
    def _():
        g = N_UNIF * N_TILES + tid
        pltpu.sync_copy(ei.at[0, g], src_vs[0])
        pltpu.sync_copy(dstf.at[pl.ds(g * CHUNK, CHUNK)], dst_vs[0])
        pltpu.sync_copy(prb.at[g], prb_vs[0])
        gather(0)
        pltpu.sync_copy(val_vs[0], acc.at[dst_vs[0]], add=True)

    plsc.subcore_barrier()

    for off, sz in _PIECES:
        pltpu.sync_copy(acc.at[pl.ds(base + off, sz)], stage.at[pl.ds(0, sz)])

        @pl.when(cid == 0)
        def _():
            pltpu.sync_copy(stage.at[pl.ds(0, sz)],
                            pa_out.at[pl.ds(base + off, sz)])

        @pl.when(cid == 1)
        def _():
            pltpu.sync_copy(stage.at[pl.ds(0, sz)],
                            pb_out.at[pl.ds(base + off, sz)])


_step = functools.partial(
    pl.kernel,
    out_type=(jax.ShapeDtypeStruct((NP,), jnp.float32),
              jax.ShapeDtypeStruct((NP,), jnp.float32)),
    mesh=_mesh,
    scratch_types=[
        pltpu.VMEM((NP,), jnp.float32),
        pltpu.VMEM((2048,), jnp.float32),
        *[pltpu.VMEM((CHUNK_R, CHUNK_C), jnp.int32)
          for _ in range(NBUF)],
        *[pltpu.VMEM((CHUNK,), jnp.int32)
          for _ in range(NBUF)],
        *[pltpu.VMEM((CHUNK_R, CHUNK_C), jnp.float32)
          for _ in range(NBUF)],
        *[pltpu.VMEM((CHUNK,), jnp.float32)
          for _ in range(NBUF)],
        *[pltpu.SemaphoreType.DMA for _ in range(NBUF)],
        pltpu.SemaphoreType.DMA,
        pltpu.VMEM_SHARED((NP,), jnp.float32),
    ],
    compiler_params=_params,
)(_step_body)


def _combine_body(w_ref, pa_ref, pb_ref, res_ref, y_out, res_out):
    y = pa_ref[...] + pb_ref[...]
    y_out[...] = y
    res_out[...] = res_ref[...] + w_ref[0] * y


def _combine(w, pa, pb, res):
    y2, r2 = pl.pallas_call(
        _combine_body,
        out_shape=(jax.ShapeDtypeStruct((NP // 128, 128), jnp.float32),
                   jax.ShapeDtypeStruct((NP // 128, 128), jnp.float32)),
        in_specs=[
            pl.BlockSpec(memory_space=pltpu.SMEM),
            pl.BlockSpec(memory_space=pltpu.VMEM),
            pl.BlockSpec(memory_space=pltpu.VMEM),
            pl.BlockSpec(memory_space=pltpu.VMEM),
        ],
    )(w, pa.reshape(NP // 128, 128), pb.reshape(NP // 128, 128),
      res.reshape(NP // 128, 128))
    return y2.reshape(NP), r2.reshape(NP)


def _fin_body(w_ref, pa_ref, pb_ref, res_ref, x0_ref, res_out):
    res_out[...] = (res_ref[...] + w_ref[0] * (pa_ref[...] + pb_ref[...])
                    + w_ref[1] * x0_ref[...])


def _fin(w2, pa, pb, res, x0):
    r2 = pl.pallas_call(
        _fin_body,
        out_shape=jax.ShapeDtypeStruct((NP // 128, 128), jnp.float32),
        in_specs=[
            pl.BlockSpec(memory_space=pltpu.SMEM),
            pl.BlockSpec(memory_space=pltpu.VMEM),
            pl.BlockSpec(memory_space=pltpu.VMEM),
            pl.BlockSpec(memory_space=pltpu.VMEM),
            pl.BlockSpec(memory_space=pltpu.VMEM),
        ],
    )(w2, pa.reshape(NP // 128, 128), pb.reshape(NP // 128, 128),
      res.reshape(NP // 128, 128), x0.reshape(NP // 128, 128))
    return r2.reshape(NP)


def kernel(x, edge_index, edge_probs, weights):
    ei = edge_index.astype(jnp.int32).reshape(2, N_CHUNKS, CHUNK_R, CHUNK_C)
    dstf = edge_index[1].astype(jnp.int32)
    prb_p = edge_probs.astype(jnp.float32).reshape(N_CHUNKS, CHUNK_R, CHUNK_C)

    x0 = jnp.pad(x[:, 0], (0, NP - N_NODES))
    zeros_np = jnp.zeros((NP,), jnp.float32)
    w = weights.astype(jnp.float32)

    y, res = x0, zeros_np
    for k in range(1, N_STEPS - 1):
        pa, pb = _step(y, ei, dstf, prb_p)
        y, res = _combine(w[k:k + 1], pa, pb, res)
    pa, pb = _step(y, ei, dstf, prb_p)
    res = _fin(jnp.stack([w[N_STEPS - 1], w[0]]), pa, pb, res, x0)
    return res[:N_NODES, None]


# speedup vs baseline: 1.0291x; 1.0291x over previous
import functools

import jax
import jax.numpy as jnp
from jax import lax
from jax.experimental import pallas as pl
from jax.experimental.pallas import tpu as pltpu
from jax.experimental.pallas import tpu_sc as plsc

N_NODES = 100000
N_EDGES = 6400000
N_STEPS = 5

NC = 2
NS = 16
N_TILES = NC * NS

SLICE = 6400
NP = NS * SLICE

CHUNK_R = 8
CHUNK_C = 128
CHUNK = CHUNK_R * CHUNK_C
N_CHUNKS = N_EDGES // CHUNK
N_UNIF = N_CHUNKS // N_TILES
N_EXTRA = N_CHUNKS - N_UNIF * N_TILES
NBUF = 4

_PIECES = [(0, 2048), (2048, 2048), (4096, 2048), (6144, 256)]

_mesh = plsc.VectorSubcoreMesh(core_axis_name="c", subcore_axis_name="s")
_params = pltpu.CompilerParams(needs_layout_passes=False)


def _step_body(x_in, eif, prb, pa_out, pb_out,
               x_buf, stage,
               src_v0, src_v1, src_v2, src_v3, dst_v0, dst_v1, dst_v2, dst_v3,
               prb_v0, prb_v1, prb_v2, prb_v3, val_v0, val_v1, val_v2, val_v3,
               sem_in0, sem_in1, sem_in2, sem_in3, sem_sc, acc):
    cid = lax.axis_index("c")
    sid = lax.axis_index("s")
    tid = cid * NS + sid
    base = sid * SLICE
    sems = [sem_in0, sem_in1, sem_in2, sem_in3]
    src_vs = [src_v0, src_v1, src_v2, src_v3]
    dst_vs = [dst_v0, dst_v1, dst_v2, dst_v3]
    prb_vs = [prb_v0, prb_v1, prb_v2, prb_v3]
    val_vs = [val_v0, val_v1, val_v2, val_v3]

    pltpu.sync_copy(x_in, x_buf)

    zv = jnp.zeros((16,), jnp.float32)

    def zero_body(j, _):
        stage[pl.ds(j * 16, 16)] = zv
        return _

    lax.fori_loop(0, 128, zero_body, 0)
    for off, sz in _PIECES:
        pltpu.sync_copy(stage.at[pl.ds(0, sz)], acc.at[pl.ds(base + off, sz)])
    plsc.subcore_barrier()

    def issue(j, b):
        g = tid + N_TILES * j
        pltpu.async_copy(eif.at[pl.ds(g * CHUNK, CHUNK)], src_vs[b], sems[b])
        pltpu.async_copy(eif.at[pl.ds(N_EDGES + g * CHUNK, CHUNK)], dst_vs[b],
                         sems[b])
        pltpu.async_copy(prb.at[pl.ds(g * CHUNK, CHUNK)], prb_vs[b], sems[b])

    def wait_in(j, b):
        g = tid + N_TILES * j
        pltpu.make_async_copy(eif.at[pl.ds(g * CHUNK, CHUNK)], src_vs[b],
                              sems[b]).wait()
        pltpu.make_async_copy(eif.at[pl.ds(N_EDGES + g * CHUNK, CHUNK)],
                              dst_vs[b], sems[b]).wait()
        pltpu.make_async_copy(prb.at[pl.ds(g * CHUNK, CHUNK)], prb_vs[b],
                              sems[b]).wait()

    def gather(b):
        for k in range(CHUNK // 16):
            idx = src_vs[b][pl.ds(k * 16, 16)]
            v = plsc.load_gather(x_buf, [idx]) * prb_vs[b][pl.ds(k * 16, 16)]
            val_vs[b][pl.ds(k * 16, 16)] = v

    def fire(b):
        pltpu.async_copy(val_vs[b], acc.at[dst_vs[b]], sem_sc, add=True)

    def drain(b):
        pltpu.make_async_copy(eif.at[pl.ds(0, CHUNK)], dst_vs[b],
                              sem_sc).wait()

    issue(0, 0)
    issue(1, 1)

    def loop_body(t, carry):
        for u in range(NBUF):
            j = t * NBUF + u
            wait_in(j, u)
            gather(u)
            if u <= 1:
                @pl.when(t >= 1)
                def _():
                    drain((u + 2) % NBUF)
            else:
                drain(u - 2)
            issue(j + 2, (u + 2) % NBUF)
            fire(u)
        return carry

    n_main = (N_UNIF - 3) // NBUF
    lax.fori_loop(0, n_main, loop_body, 0)
    for j in range(n_main * NBUF, N_UNIF):
        u = j % NBUF
        wait_in(j, u)
        gather(u)
        drain((u + 2) % NBUF)
        if j + 2 < N_UNIF:
            issue(j + 2, (j + 2) % NBUF)
        fire(u)
    drain((N_UNIF - 2) % NBUF)
    drain((N_UNIF - 1) % NBUF)

    @pl.when(tid < N_EXTRA)
    def _():
        g = N_UNIF * N_TILES + tid
        pltpu.sync_copy(eif.at[pl.ds(g * CHUNK, CHUNK)], src_vs[0])
        pltpu.sync_copy(eif.at[pl.ds(N_EDGES + g * CHUNK, CHUNK)], dst_vs[0])
        pltpu.sync_copy(prb.at[pl.ds(g * CHUNK, CHUNK)], prb_vs[0])
        gather(0)
        pltpu.sync_copy(val_vs[0], acc.at[dst_vs[0]], add=True)

    plsc.subcore_barrier()

    for off, sz in _PIECES:
        pltpu.sync_copy(acc.at[pl.ds(base + off, sz)], stage.at[pl.ds(0, sz)])

        @pl.when(cid == 0)
        def _():
            pltpu.sync_copy(stage.at[pl.ds(0, sz)],
                            pa_out.at[pl.ds(base + off, sz)])

        @pl.when(cid == 1)
        def _():
            pltpu.sync_copy(stage.at[pl.ds(0, sz)],
                            pb_out.at[pl.ds(base + off, sz)])


_step = functools.partial(
    pl.kernel,
    out_type=(jax.ShapeDtypeStruct((NP,), jnp.float32),
              jax.ShapeDtypeStruct((NP,), jnp.float32)),
    mesh=_mesh,
    scratch_types=[
        pltpu.VMEM((NP,), jnp.float32),
        pltpu.VMEM((2048,), jnp.float32),
        *[pltpu.VMEM((CHUNK,), jnp.int32)
          for _ in range(2 * NBUF)],
        *[pltpu.VMEM((CHUNK,), jnp.float32)
          for _ in range(2 * NBUF)],
        *[pltpu.SemaphoreType.DMA for _ in range(NBUF)],
        pltpu.SemaphoreType.DMA,
        pltpu.VMEM_SHARED((NP,), jnp.float32),
    ],
    compiler_params=_params,
)(_step_body)


def _combine_body(w_ref, pa_ref, pb_ref, res_ref, y_out, res_out):
    y = pa_ref[...] + pb_ref[...]
    y_out[...] = y
    res_out[...] = res_ref[...] + w_ref[0] * y


def _combine(w, pa, pb, res):
    y2, r2 = pl.pallas_call(
        _combine_body,
        out_shape=(jax.ShapeDtypeStruct((NP // 128, 128), jnp.float32),
                   jax.ShapeDtypeStruct((NP // 128, 128), jnp.float32)),
        in_specs=[
            pl.BlockSpec(memory_space=pltpu.SMEM),
            pl.BlockSpec(memory_space=pltpu.VMEM),
            pl.BlockSpec(memory_space=pltpu.VMEM),
            pl.BlockSpec(memory_space=pltpu.VMEM),
        ],
    )(w, pa.reshape(NP // 128, 128), pb.reshape(NP // 128, 128),
      res.reshape(NP // 128, 128))
    return y2.reshape(NP), r2.reshape(NP)


def _fin_body(w_ref, pa_ref, pb_ref, res_ref, x0_ref, res_out):
    res_out[...] = (res_ref[...] + w_ref[0] * (pa_ref[...] + pb_ref[...])
                    + w_ref[1] * x0_ref[...])


def _fin(w2, pa, pb, res, x0):
    r2 = pl.pallas_call(
        _fin_body,
        out_shape=jax.ShapeDtypeStruct((NP // 128, 128), jnp.float32),
        in_specs=[
            pl.BlockSpec(memory_space=pltpu.SMEM),
            pl.BlockSpec(memory_space=pltpu.VMEM),
            pl.BlockSpec(memory_space=pltpu.VMEM),
            pl.BlockSpec(memory_space=pltpu.VMEM),
            pl.BlockSpec(memory_space=pltpu.VMEM),
        ],
    )(w2, pa.reshape(NP // 128, 128), pb.reshape(NP // 128, 128),
      res.reshape(NP // 128, 128), x0.reshape(NP // 128, 128))
    return r2.reshape(NP)


def kernel(x, edge_index, edge_probs, weights):
    eif = edge_index.astype(jnp.int32).reshape(2 * N_EDGES)
    prb_p = edge_probs.astype(jnp.float32)

    x0 = jnp.pad(x[:, 0], (0, NP - N_NODES))
    zeros_np = jnp.zeros((NP,), jnp.float32)
    w = weights.astype(jnp.float32)

    y, res = x0, zeros_np
    for k in range(1, N_STEPS - 1):
        pa, pb = _step(y, eif, prb_p)
        y, res = _combine(w[k:k + 1], pa, pb, res)
    pa, pb = _step(y, eif, prb_p)
    res = _fin(jnp.stack([w[N_STEPS - 1], w[0]]), pa, pb, res, x0)
    return res[:N_NODES, None]


# speedup vs baseline: 1.0304x; 1.0013x over previous
import functools

import jax
import jax.numpy as jnp
from jax import lax
from jax.experimental import pallas as pl
from jax.experimental.pallas import tpu as pltpu
from jax.experimental.pallas import tpu_sc as plsc

N_NODES = 100000
N_EDGES = 6400000
N_STEPS = 5

NC = 2
NS = 16
N_TILES = NC * NS

SLICE = 6400
NP = NS * SLICE

CHUNK_R = 8
CHUNK_C = 128
CHUNK = CHUNK_R * CHUNK_C
N_CHUNKS = N_EDGES // CHUNK
N_UNIF = N_CHUNKS // N_TILES
N_EXTRA = N_CHUNKS - N_UNIF * N_TILES
NBUF = 4

_PIECES = [(0, 2048), (2048, 2048), (4096, 2048), (6144, 256)]

_mesh = plsc.VectorSubcoreMesh(core_axis_name="c", subcore_axis_name="s")
_params = pltpu.CompilerParams(needs_layout_passes=False)


def _step_body(x_in, ei2, prb, pa_out, pb_out,
               x_buf, stage,
               sd_v0, sd_v1, sd_v2, sd_v3,
               prb_v0, prb_v1, prb_v2, prb_v3, val_v0, val_v1, val_v2, val_v3,
               sem_in0, sem_in1, sem_in2, sem_in3, sem_sc, acc):
    cid = lax.axis_index("c")
    sid = lax.axis_index("s")
    tid = cid * NS + sid
    base = sid * SLICE
    sems = [sem_in0, sem_in1, sem_in2, sem_in3]
    sd_vs = [sd_v0, sd_v1, sd_v2, sd_v3]
    prb_vs = [prb_v0, prb_v1, prb_v2, prb_v3]
    val_vs = [val_v0, val_v1, val_v2, val_v3]

    pltpu.sync_copy(x_in, x_buf)

    zv = jnp.zeros((16,), jnp.float32)

    def zero_body(j, _):
        stage[pl.ds(j * 16, 16)] = zv
        return _

    lax.fori_loop(0, 128, zero_body, 0)
    for off, sz in _PIECES:
        pltpu.sync_copy(stage.at[pl.ds(0, sz)], acc.at[pl.ds(base + off, sz)])
    plsc.subcore_barrier()

    def issue(j, b):
        g = tid + N_TILES * j
        pltpu.async_copy(ei2.at[:, g], sd_vs[b], sems[b])
        pltpu.async_copy(prb.at[pl.ds(g * CHUNK, CHUNK)], prb_vs[b], sems[b])

    def wait_in(j, b):
        g = tid + N_TILES * j
        pltpu.make_async_copy(ei2.at[:, g], sd_vs[b], sems[b]).wait()
        pltpu.make_async_copy(prb.at[pl.ds(g * CHUNK, CHUNK)], prb_vs[b],
                              sems[b]).wait()

    def gather(b):
        for r in range(CHUNK_R):
            for j in range(CHUNK_C // 16):
                idx = sd_vs[b][0, r, pl.ds(j * 16, 16)]
                v = (plsc.load_gather(x_buf, [idx])
                     * prb_vs[b][pl.ds(r * CHUNK_C + j * 16, 16)])
                val_vs[b][pl.ds(r * CHUNK_C + j * 16, 16)] = v

    def fire(b):
        for r in range(CHUNK_R):
            pltpu.async_copy(val_vs[b].at[pl.ds(r * CHUNK_C, CHUNK_C)],
                             acc.at[sd_vs[b].at[1, r]], sem_sc, add=True)

    def drain(b):
        pltpu.make_async_copy(x_in.at[pl.ds(0, CHUNK)], val_vs[b],
                              sem_sc).wait()

    issue(0, 0)
    issue(1, 1)

    def loop_body(t, carry):
        for u in range(NBUF):
            j = t * NBUF + u
            wait_in(j, u)
            gather(u)
            if u <= 1:
                @pl.when(t >= 1)
                def _():
                    drain((u + 2) % NBUF)
            else:
                drain(u - 2)
            issue(j + 2, (u + 2) % NBUF)
            fire(u)
        return carry

    n_main = (N_UNIF - 3) // NBUF
    lax.fori_loop(0, n_main, loop_body, 0)
    for j in range(n_main * NBUF, N_UNIF):
        u = j % NBUF
        wait_in(j, u)
        gather(u)
        drain((u + 2) % NBUF)
        if j + 2 < N_UNIF:
            issue(j + 2, (j + 2) % NBUF)
        fire(u)
    drain((N_UNIF - 2) % NBUF)
    drain((N_UNIF - 1) % NBUF)

    @pl.when(tid < N_EXTRA)
    def _():
        g = N_UNIF * N_TILES + tid
        pltpu.sync_copy(ei2.at[:, g], sd_vs[0])
        pltpu.sync_copy(prb.at[pl.ds(g * CHUNK, CHUNK)], prb_vs[0])
        gather(0)
        for r in range(CHUNK_R):
            pltpu.sync_copy(val_vs[0].at[pl.ds(r * CHUNK_C, CHUNK_C)],
                            acc.at[sd_vs[0].at[1, r]], add=True)

    plsc.subcore_barrier()

    for off, sz in _PIECES:
        pltpu.sync_copy(acc.at[pl.ds(base + off, sz)], stage.at[pl.ds(0, sz)])

        @pl.when(cid == 0)
        def _():
            pltpu.sync_copy(stage.at[pl.ds(0, sz)],
                            pa_out.at[pl.ds(base + off, sz)])

        @pl.when(cid == 1)
        def _():
            pltpu.sync_copy(stage.at[pl.ds(0, sz)],
                            pb_out.at[pl.ds(base + off, sz)])


_step = functools.partial(
    pl.kernel,
    out_type=(jax.ShapeDtypeStruct((NP,), jnp.float32),
              jax.ShapeDtypeStruct((NP,), jnp.float32)),
    mesh=_mesh,
    scratch_types=[
        pltpu.VMEM((NP,), jnp.float32),
        pltpu.VMEM((2048,), jnp.float32),
        *[pltpu.VMEM((2, CHUNK_R, CHUNK_C), jnp.int32)
          for _ in range(NBUF)],
        *[pltpu.VMEM((CHUNK,), jnp.float32)
          for _ in range(NBUF)],
        *[pltpu.VMEM((CHUNK,), jnp.float32)
          for _ in range(NBUF)],
        *[pltpu.SemaphoreType.DMA for _ in range(NBUF)],
        pltpu.SemaphoreType.DMA,
        pltpu.VMEM_SHARED((NP,), jnp.float32),
    ],
    compiler_params=_params,
)(_step_body)


def _combine_body(w_ref, pa_ref, pb_ref, res_ref, y_out, res_out):
    y = pa_ref[...] + pb_ref[...]
    y_out[...] = y
    res_out[...] = res_ref[...] + w_ref[0] * y


def _combine(w, pa, pb, res):
    y2, r2 = pl.pallas_call(
        _combine_body,
        out_shape=(jax.ShapeDtypeStruct((NP // 128, 128), jnp.float32),
                   jax.ShapeDtypeStruct((NP // 128, 128), jnp.float32)),
        in_specs=[
            pl.BlockSpec(memory_space=pltpu.SMEM),
            pl.BlockSpec(memory_space=pltpu.VMEM),
            pl.BlockSpec(memory_space=pltpu.VMEM),
            pl.BlockSpec(memory_space=pltpu.VMEM),
        ],
    )(w, pa.reshape(NP // 128, 128), pb.reshape(NP // 128, 128),
      res.reshape(NP // 128, 128))
    return y2.reshape(NP), r2.reshape(NP)


def _fin_body(w_ref, pa_ref, pb_ref, res_ref, x0_ref, res_out):
    res_out[...] = (res_ref[...] + w_ref[0] * (pa_ref[...] + pb_ref[...])
                    + w_ref[1] * x0_ref[...])


def _fin(w2, pa, pb, res, x0):
    r2 = pl.pallas_call(
        _fin_body,
        out_shape=jax.ShapeDtypeStruct((NP // 128, 128), jnp.float32),
        in_specs=[
            pl.BlockSpec(memory_space=pltpu.SMEM),
            pl.BlockSpec(memory_space=pltpu.VMEM),
            pl.BlockSpec(memory_space=pltpu.VMEM),
            pl.BlockSpec(memory_space=pltpu.VMEM),
            pl.BlockSpec(memory_space=pltpu.VMEM),
        ],
    )(w2, pa.reshape(NP // 128, 128), pb.reshape(NP // 128, 128),
      res.reshape(NP // 128, 128), x0.reshape(NP // 128, 128))
    return r2.reshape(NP)


def kernel(x, edge_index, edge_probs, weights):
    ei2 = edge_index.astype(jnp.int32).reshape(2, N_CHUNKS, CHUNK_R, CHUNK_C)
    prb_p = edge_probs.astype(jnp.float32)

    x0 = jnp.pad(x[:, 0], (0, NP - N_NODES))
    zeros_np = jnp.zeros((NP,), jnp.float32)
    w = weights.astype(jnp.float32)

    y, res = x0, zeros_np
    for k in range(1, N_STEPS - 1):
        pa, pb = _step(y, ei2, prb_p)
        y, res = _combine(w[k:k + 1], pa, pb, res)
    pa, pb = _step(y, ei2, prb_p)
    res = _fin(jnp.stack([w[N_STEPS - 1], w[0]]), pa, pb, res, x0)
    return res[:N_NODES, None]


# speedup vs baseline: 1.0780x; 1.0462x over previous
import functools

import jax
import jax.numpy as jnp
from jax import lax
from jax.experimental import pallas as pl
from jax.experimental.pallas import tpu as pltpu
from jax.experimental.pallas import tpu_sc as plsc

N_NODES = 100000
N_EDGES = 6400000
N_STEPS = 5

NC = 2
NS = 16
N_TILES = NC * NS

SLICE = 6400
NP = NS * SLICE

CHUNK_R = 8
CHUNK_C = 128
CHUNK = CHUNK_R * CHUNK_C
N_CHUNKS = N_EDGES // CHUNK
N_UNIF = N_CHUNKS // N_TILES
N_EXTRA = N_CHUNKS - N_UNIF * N_TILES
NBUF = 5

_PIECES = [(i * 1024, 1024) for i in range(6)] + [(6144, 256)]

_mesh = plsc.VectorSubcoreMesh(core_axis_name="c", subcore_axis_name="s")
_params = pltpu.CompilerParams(needs_layout_passes=False)


def _step_body(x_in, ei2, prb, pa_out, pb_out,
               x_buf, stage,
               sd_v0, sd_v1, sd_v2, sd_v3, sd_v4,
               prb_v0, prb_v1, prb_v2, prb_v3, prb_v4,
               val_v0, val_v1, val_v2, val_v3, val_v4,
               sem_in0, sem_in1, sem_in2, sem_in3, sem_in4, sem_sc, acc):
    cid = lax.axis_index("c")
    sid = lax.axis_index("s")
    tid = cid * NS + sid
    base = sid * SLICE
    sems = [sem_in0, sem_in1, sem_in2, sem_in3, sem_in4]
    sd_vs = [sd_v0, sd_v1, sd_v2, sd_v3, sd_v4]
    prb_vs = [prb_v0, prb_v1, prb_v2, prb_v3, prb_v4]
    val_vs = [val_v0, val_v1, val_v2, val_v3, val_v4]

    pltpu.sync_copy(x_in, x_buf)

    zv = jnp.zeros((16,), jnp.float32)

    def zero_body(j, _):
        stage[pl.ds(j * 16, 16)] = zv
        return _

    lax.fori_loop(0, 64, zero_body, 0)
    for off, sz in _PIECES:
        pltpu.sync_copy(stage.at[pl.ds(0, sz)], acc.at[pl.ds(base + off, sz)])
    plsc.subcore_barrier()

    def issue(j, b):
        g = tid + N_TILES * j
        pltpu.async_copy(ei2.at[:, g], sd_vs[b], sems[b])
        pltpu.async_copy(prb.at[pl.ds(g * CHUNK, CHUNK)], prb_vs[b], sems[b])

    def wait_in(j, b):
        g = tid + N_TILES * j
        pltpu.make_async_copy(ei2.at[:, g], sd_vs[b], sems[b]).wait()
        pltpu.make_async_copy(prb.at[pl.ds(g * CHUNK, CHUNK)], prb_vs[b],
                              sems[b]).wait()

    def gather(b):
        for r in range(CHUNK_R):
            for j in range(CHUNK_C // 16):
                idx = sd_vs[b][0, r, pl.ds(j * 16, 16)]
                v = (plsc.load_gather(x_buf, [idx])
                     * prb_vs[b][pl.ds(r * CHUNK_C + j * 16, 16)])
                val_vs[b][pl.ds(r * CHUNK_C + j * 16, 16)] = v

    def fire(b):
        for r in range(CHUNK_R):
            pltpu.async_copy(val_vs[b].at[pl.ds(r * CHUNK_C, CHUNK_C)],
                             acc.at[sd_vs[b].at[1, r]], sem_sc, add=True)

    def drain(b):
        pltpu.make_async_copy(x_in.at[pl.ds(0, CHUNK)], val_vs[b],
                              sem_sc).wait()

    issue(0, 0)
    issue(1, 1)
    issue(2, 2)

    def loop_body(t, carry):
        for u in range(NBUF):
            j = t * NBUF + u
            wait_in(j, u)
            gather(u)
            if u <= 1:
                @pl.when(t >= 1)
                def _():
                    drain((u - 2) % NBUF)
            else:
                drain(u - 2)
            issue(j + 3, (u + 3) % NBUF)
            fire(u)
        return carry

    n_main = (N_UNIF - 4) // NBUF
    lax.fori_loop(0, n_main, loop_body, 0)
    for j in range(n_main * NBUF, N_UNIF):
        u = j % NBUF
        wait_in(j, u)
        gather(u)
        drain((u - 2) % NBUF)
        if j + 3 < N_UNIF:
            issue(j + 3, (j + 3) % NBUF)
        fire(u)
    drain((N_UNIF - 2) % NBUF)
    drain((N_UNIF - 1) % NBUF)

    @pl.when(tid < N_EXTRA)
    def _():
        g = N_UNIF * N_TILES + tid
        pltpu.sync_copy(ei2.at[:, g], sd_vs[0])
        pltpu.sync_copy(prb.at[pl.ds(g * CHUNK, CHUNK)], prb_vs[0])
        gather(0)
        for r in range(CHUNK_R):
            pltpu.sync_copy(val_vs[0].at[pl.ds(r * CHUNK_C, CHUNK_C)],
                            acc.at[sd_vs[0].at[1, r]], add=True)

    plsc.subcore_barrier()

    for off, sz in _PIECES:
        pltpu.sync_copy(acc.at[pl.ds(base + off, sz)], stage.at[pl.ds(0, sz)])

        @pl.when(cid == 0)
        def _():
            pltpu.sync_copy(stage.at[pl.ds(0, sz)],
                            pa_out.at[pl.ds(base + off, sz)])

        @pl.when(cid == 1)
        def _():
            pltpu.sync_copy(stage.at[pl.ds(0, sz)],
                            pb_out.at[pl.ds(base + off, sz)])


_step = functools.partial(
    pl.kernel,
    out_type=(jax.ShapeDtypeStruct((NP,), jnp.float32),
              jax.ShapeDtypeStruct((NP,), jnp.float32)),
    mesh=_mesh,
    scratch_types=[
        pltpu.VMEM((NP,), jnp.float32),
        pltpu.VMEM((1024,), jnp.float32),
        *[pltpu.VMEM((2, CHUNK_R, CHUNK_C), jnp.int32)
          for _ in range(NBUF)],
        *[pltpu.VMEM((CHUNK,), jnp.float32)
          for _ in range(NBUF)],
        *[pltpu.VMEM((CHUNK,), jnp.float32)
          for _ in range(NBUF)],
        *[pltpu.SemaphoreType.DMA for _ in range(NBUF)],
        pltpu.SemaphoreType.DMA,
        pltpu.VMEM_SHARED((NP,), jnp.float32),
    ],
    compiler_params=_params,
)(_step_body)


def _combine_body(w_ref, pa_ref, pb_ref, res_ref, y_out, res_out):
    y = pa_ref[...] + pb_ref[...]
    y_out[...] = y
    res_out[...] = res_ref[...] + w_ref[0] * y


def _combine(w, pa, pb, res):
    y2, r2 = pl.pallas_call(
        _combine_body,
        out_shape=(jax.ShapeDtypeStruct((NP // 128, 128), jnp.float32),
                   jax.ShapeDtypeStruct((NP // 128, 128), jnp.float32)),
        in_specs=[
            pl.BlockSpec(memory_space=pltpu.SMEM),
            pl.BlockSpec(memory_space=pltpu.VMEM),
            pl.BlockSpec(memory_space=pltpu.VMEM),
            pl.BlockSpec(memory_space=pltpu.VMEM),
        ],
    )(w, pa.reshape(NP // 128, 128), pb.reshape(NP // 128, 128),
      res.reshape(NP // 128, 128))
    return y2.reshape(NP), r2.reshape(NP)


def _fin_body(w_ref, pa_ref, pb_ref, res_ref, x0_ref, res_out):
    res_out[...] = (res_ref[...] + w_ref[0] * (pa_ref[...] + pb_ref[...])
                    + w_ref[1] * x0_ref[...])


def _fin(w2, pa, pb, res, x0):
    r2 = pl.pallas_call(
        _fin_body,
        out_shape=jax.ShapeDtypeStruct((NP // 128, 128), jnp.float32),
        in_specs=[
            pl.BlockSpec(memory_space=pltpu.SMEM),
            pl.BlockSpec(memory_space=pltpu.VMEM),
            pl.BlockSpec(memory_space=pltpu.VMEM),
            pl.BlockSpec(memory_space=pltpu.VMEM),
            pl.BlockSpec(memory_space=pltpu.VMEM),
        ],
    )(w2, pa.reshape(NP // 128, 128), pb.reshape(NP // 128, 128),
      res.reshape(NP // 128, 128), x0.reshape(NP // 128, 128))
    return r2.reshape(NP)


def kernel(x, edge_index, edge_probs, weights):
    ei2 = edge_index.astype(jnp.int32).reshape(2, N_CHUNKS, CHUNK_R, CHUNK_C)
    prb_p = edge_probs.astype(jnp.float32)

    x0 = jnp.pad(x[:, 0], (0, NP - N_NODES))
    zeros_np = jnp.zeros((NP,), jnp.float32)
    w = weights.astype(jnp.float32)

    y, res = x0, zeros_np
    for k in range(1, N_STEPS - 1):
        pa, pb = _step(y, ei2, prb_p)
        y, res = _combine(w[k:k + 1], pa, pb, res)
    pa, pb = _step(y, ei2, prb_p)
    res = _fin(jnp.stack([w[N_STEPS - 1], w[0]]), pa, pb, res, x0)
    return res[:N_NODES, None]
